# Initial kernel scaffold; baseline (speedup 1.0000x reference)
import jax
import jax.numpy as jnp
from jax.experimental import pallas as pl


def kernel(neighbour, W, b):
    raise NotImplementedError("write your pallas kernel here")


# speedup vs baseline: 1.1290x; 1.1290x over previous
import jax
import jax.numpy as jnp
from jax.experimental import pallas as pl

BN = 400


def _agg_kernel(x_ref, wt_ref, b_ref, out_ref):
    bn = x_ref.shape[0]
    deg = x_ref.shape[1]
    x = x_ref[...].reshape(bn * deg, x_ref.shape[2])
    h = jnp.dot(x, wt_ref[...], preferred_element_type=jnp.float32)
    hr = h.reshape(bn, deg, h.shape[1])
    out_ref[...] = jnp.max(hr, axis=1) + b_ref[...]


def kernel(neighbour, W, b):
    n, deg, d_in = neighbour.shape
    d_out = W.shape[0]
    wt = W.T
    b2 = b.reshape(1, d_out)
    grid = (n // BN,)
    return pl.pallas_call(
        _agg_kernel,
        grid=grid,
        in_specs=[
            pl.BlockSpec((BN, deg, d_in), lambda i: (i, 0, 0)),
            pl.BlockSpec((d_in, d_out), lambda i: (0, 0)),
            pl.BlockSpec((1, d_out), lambda i: (0, 0)),
        ],
        out_specs=pl.BlockSpec((BN, d_out), lambda i: (i, 0)),
        out_shape=jax.ShapeDtypeStruct((n, d_out), jnp.float32),
    )(neighbour, wt, b2)
